# Initial kernel scaffold; baseline (speedup 1.0000x reference)
import jax
import jax.numpy as jnp
from jax.experimental import pallas as pl


def kernel(x, W1, al1, ar1, W2, al2, ar2, Wg, bg, Wf1, bf1, Wf2, bf2, edge_index):
    raise NotImplementedError("write your pallas kernel here")


# speedup vs baseline: 35.8666x; 35.8666x over previous
import functools
import jax
import jax.numpy as jnp
from jax import lax
from jax.experimental import pallas as pl
from jax.experimental.pallas import tpu as pltpu
from jax.experimental.pallas import tpu_sc as plsc

_N = 10000
_E = 320000
_H = 64

_NC = 2
_NS = 16
_NW = _NC * _NS
_EPW = _E // _NW
_C = 80
_NCH = _EPW // _C
_ZR = 80
_NB = _N // _ZR

_SELU_ALPHA = 1.6732632423543772
_SELU_SCALE = 1.0507009873554805


def _selu(x):
    return _SELU_SCALE * jnp.where(x > 0, x, _SELU_ALPHA * (jnp.exp(x) - 1.0))


def _leaky(x):
    return jnp.where(x >= 0, x, 0.2 * x)


def _prep_body(x_ref, w_ref, al_ref, ar_ref, h_ref, el_ref, er_ref, m_ref):
    h = jnp.dot(x_ref[...], w_ref[...], preferred_element_type=jnp.float32)
    h_ref[...] = h
    el = jnp.dot(h, al_ref[...], preferred_element_type=jnp.float32)
    er = jnp.dot(h, ar_ref[...], preferred_element_type=jnp.float32)
    el_ref[...] = el
    er_ref[...] = er
    m_ref[...] = jnp.reshape(_leaky(jnp.max(el) + jnp.max(er)), (1, 1))


def _prep(x, w, al2, ar2):
    d = x.shape[1]
    return pl.pallas_call(
        _prep_body,
        out_shape=[
            jax.ShapeDtypeStruct((_N, _H), jnp.float32),
            jax.ShapeDtypeStruct((_N, 1), jnp.float32),
            jax.ShapeDtypeStruct((_N, 1), jnp.float32),
            jax.ShapeDtypeStruct((1, 1), jnp.float32),
        ],
    )(x, w, al2, ar2)


def _mid_body(num_ref, den_ref, w_ref, al_ref, ar_ref,
              h_ref, el_ref, er_ref, m_ref):
    num = num_ref[0] + num_ref[1]
    den = jnp.sum(den_ref[...], axis=0)
    hin = _selu(num / (den[:, None] + 1e-9))
    h = jnp.dot(hin, w_ref[...], preferred_element_type=jnp.float32)
    h_ref[...] = h
    el = jnp.dot(h, al_ref[...], preferred_element_type=jnp.float32)
    er = jnp.dot(h, ar_ref[...], preferred_element_type=jnp.float32)
    el_ref[...] = el
    er_ref[...] = er
    m_ref[...] = jnp.reshape(_leaky(jnp.max(el) + jnp.max(er)), (1, 1))


def _mid(num, den, w, al2, ar2):
    return pl.pallas_call(
        _mid_body,
        out_shape=[
            jax.ShapeDtypeStruct((_N, _H), jnp.float32),
            jax.ShapeDtypeStruct((_N, 1), jnp.float32),
            jax.ShapeDtypeStruct((_N, 1), jnp.float32),
            jax.ShapeDtypeStruct((1, 1), jnp.float32),
        ],
    )(num, den, w, al2, ar2)


def _head_body(num_ref, den_ref, wg_ref, bg_ref, wf1_ref, bf1_ref,
               wf2_ref, bf2_ref, out_ref):
    num = num_ref[0] + num_ref[1]
    den = jnp.sum(den_ref[...], axis=0)
    h = _selu(num / (den[:, None] + 1e-9))
    gate = jnp.dot(h, wg_ref[...],
                   preferred_element_type=jnp.float32) + bg_ref[0, 0]
    gmax = jnp.max(gate)
    eg = jnp.exp(gate - gmax)
    attn = eg / jnp.sum(eg)
    readout = jnp.sum(attn * h, axis=0, keepdims=True)
    z = _selu(jnp.dot(readout, wf1_ref[...],
                      preferred_element_type=jnp.float32) + bf1_ref[...])
    y = jnp.dot(z, wf2_ref[...],
                preferred_element_type=jnp.float32) + bf2_ref[...]
    out_ref[...] = 1.0 / (1.0 + jnp.exp(-y))


def _head(num, den, wg, bg2, wf1, bf12, wf2, bf22):
    return pl.pallas_call(
        _head_body,
        out_shape=jax.ShapeDtypeStruct((1, 1), jnp.float32),
    )(num, den, wg, bg2, wf1, bf12, wf2, bf22)


_MESH = plsc.VectorSubcoreMesh(core_axis_name="c", subcore_axis_name="s")


@functools.partial(
    pl.kernel,
    out_type=[
        jax.ShapeDtypeStruct((_NC, _N, _H), jnp.float32),
        jax.ShapeDtypeStruct((_NW, _N), jnp.float32),
    ],
    mesh=_MESH,
    compiler_params=pltpu.CompilerParams(needs_layout_passes=False,
                                         use_tc_tiling_on_sc=False),
    scratch_types=[
        pltpu.VMEM((_N,), jnp.float32),
        pltpu.VMEM((_N,), jnp.float32),
        pltpu.VMEM((_EPW,), jnp.int32),
        pltpu.VMEM((_NCH, _C), jnp.int32),
        pltpu.VMEM((_N,), jnp.float32),
        pltpu.VMEM((_C,), jnp.float32),
        pltpu.VMEM((_C, _H), jnp.float32),
        pltpu.VMEM((_ZR, _H), jnp.float32),
        pltpu.VMEM((16,), jnp.float32),
        pltpu.SemaphoreType.DMA,
        pltpu.VMEM_SHARED((_N, _H), jnp.float32),
    ],
)
def _edge_pass(h_hbm, el_hbm, er_hbm, src_hbm, dst3_hbm, m_hbm,
               num_out, den_out,
               el_v, er_v, src_v, dst2d, den_v, w_v, rows_v, z_v,
               m_v, sem, acc):
    c = lax.axis_index("c")
    s = lax.axis_index("s")
    wid = c * _NS + s

    pltpu.sync_copy(el_hbm, el_v)
    pltpu.sync_copy(er_hbm, er_v)
    pltpu.sync_copy(m_hbm, m_v)
    pltpu.sync_copy(src_hbm.at[wid], src_v)
    pltpu.sync_copy(dst3_hbm.at[wid], dst2d)

    zeros = jnp.zeros((16,), jnp.float32)

    def _z1(i, _):
        for k in range(_H // 16):
            z_v[i, pl.ds(k * 16, 16)] = zeros
        return 0
    lax.fori_loop(0, _ZR, _z1, 0, unroll=4)

    def _z2(i, _):
        den_v[pl.ds(i * 16, 16)] = zeros
        return 0
    lax.fori_loop(0, _N // 16, _z2, 0, unroll=4)

    def _z3(k, _):
        j = k * _NS + s

        @pl.when(j < _NB)
        def _():
            pltpu.sync_copy(z_v, acc.at[pl.ds(j * _ZR, _ZR)])
        return 0
    lax.fori_loop(0, (_NB + _NS - 1) // _NS, _z3, 0)

    plsc.subcore_barrier()

    m16 = m_v[...]

    def _chunk(ci, _):
        off = ci * _C
        cop = pltpu.async_copy(h_hbm.at[src_v.at[pl.ds(off, _C)]], rows_v, sem)

        for i in range(_C // 16):
            s16 = src_v[pl.ds(off + i * 16, 16)]
            d16 = dst2d[ci, pl.ds(i * 16, 16)]
            a = plsc.load_gather(el_v, [s16])
            b = plsc.load_gather(er_v, [d16])
            w = jnp.exp(_leaky(a + b) - m16)
            w_v[pl.ds(i * 16, 16)] = w
            plsc.addupdate_scatter(den_v, [d16], w)

        cop.wait()

        def _scale(r, _):
            ws = plsc.load_gather(w_v, [jnp.full((16,), r, jnp.int32)])
            for k in range(_H // 16):
                v = rows_v[r, pl.ds(k * 16, 16)]
                rows_v[r, pl.ds(k * 16, 16)] = v * ws
            return 0
        lax.fori_loop(0, _C, _scale, 0)

        pltpu.sync_copy(rows_v, acc.at[dst2d.at[ci]], add=True)
        return 0

    lax.fori_loop(0, _NCH, _chunk, 0)

    plsc.subcore_barrier()

    pltpu.sync_copy(den_v, den_out.at[wid])

    def _dump(k, _):
        j = k * _NS + s

        @pl.when(j < _NB)
        def _():
            o = j * _ZR
            pltpu.sync_copy(acc.at[pl.ds(o, _ZR)],
                            num_out.at[c, pl.ds(o, _ZR)])
        return 0
    lax.fori_loop(0, (_NB + _NS - 1) // _NS, _dump, 0)


def kernel(x, W1, al1, ar1, W2, al2, ar2, Wg, bg, Wf1, bf1, Wf2, bf2,
           edge_index):
    src = edge_index[0].reshape(_NW, _EPW)
    dst3 = edge_index[1].reshape(_NW, _NCH, _C)

    h1, el1, er1, m1 = _prep(x, W1, al1.reshape(_H, 1), ar1.reshape(_H, 1))
    m1v = jnp.broadcast_to(m1.reshape(()), (16,))
    num1, den1 = _edge_pass(h1, el1.reshape(_N), er1.reshape(_N),
                            src, dst3, m1v)

    h2, el2, er2, m2 = _mid(num1, den1, W2,
                            al2.reshape(_H, 1), ar2.reshape(_H, 1))
    m2v = jnp.broadcast_to(m2.reshape(()), (16,))
    num2, den2 = _edge_pass(h2, el2.reshape(_N), er2.reshape(_N),
                            src, dst3, m2v)

    return _head(num2, den2, Wg, bg.reshape(1, 1), Wf1,
                 bf1.reshape(1, _H // 2), Wf2, bf2.reshape(1, 1))
